# baseline (speedup 1.0000x reference)
import functools

import jax
import jax.numpy as jnp
from jax import lax
from jax.experimental import pallas as pl
from jax.experimental.pallas import tpu as pltpu
from jax.experimental.pallas import tpu_sc as plsc

N, D, G = 10000, 128, 16
NC, NS = 2, 16
NW = NC * NS
CHUNK = -(-N // NW // 16) * 16
LASTC = N - (NW - 1) * CHUNK
NGRP = CHUNK // 16
LGRP = LASTC // 16
NJ = D // 16


def _sc_partial_segsum(xf, ids):
    mesh = plsc.VectorSubcoreMesh(core_axis_name="c", subcore_axis_name="s",
                                  num_cores=NC, num_subcores=NS)

    @functools.partial(
        pl.kernel,
        out_type=(
            jax.ShapeDtypeStruct((NW, G * D), jnp.float32),
            jax.ShapeDtypeStruct((NW, G), jnp.float32),
        ),
        mesh=mesh,
        scratch_types=[
            pltpu.VMEM((CHUNK * D,), jnp.float32),
            pltpu.VMEM((CHUNK,), jnp.int32),
            pltpu.VMEM((G * D,), jnp.float32),
            pltpu.VMEM((G,), jnp.float32),
        ],
    )
    def k(x_hbm, ids_hbm, pacc_hbm, pcnt_hbm, x_v, ids_v, acc_v, cnt_v):
        wid = lax.axis_index("s") * NC + lax.axis_index("c")
        base = wid * CHUNK
        is_last = wid == NW - 1

        PIECE = 320 * D

        @pl.when(jnp.logical_not(is_last))
        def _():
            for p in range(0, CHUNK * D, PIECE):
                sz = min(PIECE, CHUNK * D - p)
                pltpu.sync_copy(x_hbm.at[pl.ds(base * D + p, sz)],
                                x_v.at[pl.ds(p, sz)])
            pltpu.sync_copy(ids_hbm.at[pl.ds(base, CHUNK)],
                            ids_v.at[pl.ds(0, CHUNK)])

        @pl.when(is_last)
        def _():
            for p in range(0, LASTC * D, PIECE):
                sz = min(PIECE, LASTC * D - p)
                pltpu.sync_copy(x_hbm.at[pl.ds((NW - 1) * CHUNK * D + p, sz)],
                                x_v.at[pl.ds(p, sz)])
            pltpu.sync_copy(ids_hbm.at[pl.ds((NW - 1) * CHUNK, LASTC)],
                            ids_v.at[pl.ds(0, LASTC)])

        zero16 = jnp.zeros((16,), jnp.float32)
        for s in range(G * NJ):
            acc_v[pl.ds(16 * s, 16)] = zero16

        ngroups = jnp.where(is_last, LGRP, NGRP)
        lane = lax.iota(jnp.int32, 16)

        def body(g, cntvec):
            idvec = ids_v[pl.ds(16 * g, 16)]
            rowbase = g * (16 * D)
            for l in range(16):
                sid = idvec[l]
                for j in range(NJ):
                    plsc.addupdate(acc_v.at[pl.ds(sid * D + 16 * j, 16)],
                                   x_v[pl.ds(rowbase + l * D + 16 * j, 16)])
                cntvec = cntvec + jnp.where(lane == sid, 1.0, 0.0)
            return cntvec

        cnt_v[...] = lax.fori_loop(0, ngroups, body, zero16)

        pltpu.sync_copy(acc_v, pacc_hbm.at[wid])
        pltpu.sync_copy(cnt_v, pcnt_hbm.at[wid])

    return k(xf, ids)


def _tc_finalize(pacc, pcnt, Wc, bc):
    def k(pacc_ref, pcnt_ref, wc_ref, bc_ref, o_ref):
        acc = pacc_ref[0]
        for i in range(1, NW):
            acc = acc + pacc_ref[i]
        ones = jnp.ones((NW, 1), jnp.float32)
        cnt = lax.dot_general(pcnt_ref[...], ones,
                              (((0,), (0,)), ((), ())),
                              preferred_element_type=jnp.float32)
        pooled = acc / jnp.maximum(cnt, 1.0)
        out = jnp.dot(pooled, wc_ref[...],
                      preferred_element_type=jnp.float32)
        o_ref[...] = out + bc_ref[...]

    return pl.pallas_call(
        k,
        out_shape=jax.ShapeDtypeStruct((G, Wc.shape[1]), jnp.float32),
    )(pacc, pcnt, Wc, bc)


def kernel(x, edge_index, edge_attr, batch_size, W1, b1, W2, b2, Wc, bc):
    pacc, pcnt = _sc_partial_segsum(x.reshape(-1), batch_size)
    pacc = pacc.reshape(NW, G, D)
    return _tc_finalize(pacc, pcnt, Wc, bc.reshape(1, -1))
